# baseline (speedup 1.0000x reference)
import functools

import numpy as np
import jax
import jax.numpy as jnp
from jax import lax
from jax.experimental import pallas as pl
from jax.experimental.pallas import tpu as pltpu
from jax.experimental.pallas import tpu_sc as plsc

B = 16384
N_COL = 26
VOCAB = 100000
D = 64
ROWS = N_COL * D

NC = 2
NS = 16
L = 16
NW = NC * NS
D_PER_W = D // NW

ALIGNED = 99968
POFF = [0, 32768, 65536, 98304]
PSZ = [32768, 32768, 32768, 1664]
NP = 4
TAIL = 128
VBUF = 32768
P3SZ = PSZ[3] + TAIL

GROUPS = B // L
UNROLL = 8


def _make_pe(max_len, d):
    position = np.arange(max_len, dtype=np.float64)[:, None]
    div_term = np.exp(np.arange(0, d, 2, dtype=np.float64) * (-np.log(10000.0) / d))
    pe = np.zeros((max_len, d), dtype=np.float32)
    pe[:, 0::2] = np.sin(position * div_term).astype(np.float32)
    pe[:, 1::2] = np.cos(position * div_term).astype(np.float32)
    return pe


_PE26 = _make_pe(N_COL, D)

_MESH = plsc.VectorSubcoreMesh(core_axis_name="c", subcore_axis_name="s")


@functools.partial(
    pl.kernel,
    mesh=_MESH,
    out_type=jax.ShapeDtypeStruct((ROWS, B), jnp.float32),
    scratch_types=[
        pltpu.VMEM((B,), jnp.int32),
        pltpu.VMEM((VBUF,), jnp.float32),
        pltpu.VMEM((VBUF,), jnp.float32),
        pltpu.VMEM((B,), jnp.float32),
        pltpu.VMEM((B,), jnp.float32),
        pltpu.SemaphoreType.DMA,
        pltpu.SemaphoreType.DMA,
        pltpu.SemaphoreType.DMA,
        pltpu.SemaphoreType.DMA,
    ],
    compiler_params=pltpu.CompilerParams(needs_layout_passes=False),
)
def _sc_gather(idx_hbm, tab_hbm, tail_hbm, out_hbm, idx_v, vb0, vb1, ov0, ov1,
               sv0, sv1, so0, so1):
    wid = lax.axis_index("s") * NC + lax.axis_index("c")
    vb = [vb0, vb1]
    ov = [ov0, ov1]
    sv = [sv0, sv1]
    so = [so0, so1]

    def vrow(c, rb):
        return c * D + wid * D_PER_W + rb

    def start_vdma(nrow, p, sb):
        pltpu.async_copy(
            tab_hbm.at[nrow].at[pl.ds(POFF[p], PSZ[p])],
            vb[sb].at[pl.ds(0, PSZ[p])],
            sv[sb],
        )
        if p == NP - 1:
            pltpu.async_copy(
                tail_hbm.at[nrow],
                vb[sb].at[pl.ds(PSZ[p], TAIL)],
                sv[sb],
            )

    def wait_vdma(p, sb):
        pltpu.make_async_copy(
            tab_hbm.at[0].at[pl.ds(POFF[p], PSZ[p])],
            vb[sb].at[pl.ds(0, PSZ[p])],
            sv[sb],
        ).wait()
        if p == NP - 1:
            pltpu.make_async_copy(
                tail_hbm.at[0],
                vb[sb].at[pl.ds(PSZ[p], TAIL)],
                sv[sb],
            ).wait()

    start_vdma(vrow(0, 0), 0, 0)
    pltpu.async_copy(ov0, out_hbm.at[vrow(0, 0)], so0)
    pltpu.async_copy(ov1, out_hbm.at[vrow(0, 1)], so1)

    def column(c, carry):
        pltpu.sync_copy(idx_hbm.at[c], idx_v)
        cn = jnp.minimum(c + 1, N_COL - 1)

        for rb in range(D_PER_W):
            row = vrow(c, rb)
            pltpu.make_async_copy(ov[rb], out_hbm.at[row], so[rb]).wait()
            for p in range(NP):
                s = rb * NP + p
                sb = s % 2
                wait_vdma(p, sb)
                if p < NP - 1:
                    start_vdma(row, p + 1, sb ^ 1)
                elif rb < D_PER_W - 1:
                    start_vdma(row + 1, 0, sb ^ 1)
                else:
                    start_vdma(vrow(cn, 0), 0, sb ^ 1)

                lo = POFF[p]
                sz = PSZ[p] if p < NP - 1 else P3SZ

                def pass_body(gg, carry2, rb=rb, p=p, sb=sb, lo=lo, sz=sz):
                    for u in range(UNROLL):
                        o = gg * (UNROLL * L) + u * L
                        iv = idx_v[pl.ds(o, L)]
                        t = iv - lo
                        ivl = jnp.minimum(jnp.maximum(t, 0), sz - 1)
                        valid = (t >= 0) & (t < sz)
                        vals = plsc.load_gather(vb[sb], [ivl])
                        contr = jnp.where(valid, vals, 0.0)
                        if p == 0:
                            ov[rb][pl.ds(o, L)] = contr
                        else:
                            plsc.addupdate(ov[rb].at[pl.ds(o, L)], contr)
                    return carry2

                lax.fori_loop(0, GROUPS // UNROLL, pass_body, 0)
            pltpu.async_copy(ov[rb], out_hbm.at[row], so[rb])
        return carry

    lax.fori_loop(0, N_COL, column, 0)

    pltpu.make_async_copy(ov0, out_hbm.at[0], so0).wait()
    pltpu.make_async_copy(ov1, out_hbm.at[0], so1).wait()
    wait_vdma(0, 0)


BC = 2048
NBC = B // BC


def _tc_body(emb_ref, w_ref, add_ref, out_ref):
    h = jnp.maximum(emb_ref[0], 0.0)
    out_ref[0] = (
        jnp.dot(w_ref[...], h, preferred_element_type=jnp.float32) + add_ref[0]
    )


def _tc_dense(emb_t, w, add3):
    return pl.pallas_call(
        _tc_body,
        grid=(N_COL, NBC),
        in_specs=[
            pl.BlockSpec((1, D, BC), lambda c, j: (c, 0, j)),
            pl.BlockSpec((D, D), lambda c, j: (0, 0)),
            pl.BlockSpec((1, D, 1), lambda c, j: (c, 0, 0)),
        ],
        out_specs=pl.BlockSpec((1, D, BC), lambda c, j: (c, 0, j)),
        out_shape=jax.ShapeDtypeStruct((N_COL, D, B), jnp.float32),
    )(emb_t, w, add3)


def kernel(x, tables, W, b):
    idx_t = x.T.astype(jnp.int32)
    tab_t = tables.transpose(0, 2, 1).reshape(ROWS, VOCAB)
    tail = jnp.pad(tab_t[:, ALIGNED:], ((0, 0), (0, TAIL - (VOCAB - ALIGNED))))
    emb_t = _sc_gather(idx_t, tab_t, tail)
    add3 = (jnp.asarray(_PE26) + b[None, :])[:, :, None]
    out_t = _tc_dense(emb_t.reshape(N_COL, D, B), W, add3)
    return out_t.transpose(2, 0, 1)


# speedup vs baseline: 1.8942x; 1.8942x over previous
import functools

import numpy as np
import jax
import jax.numpy as jnp
from jax import lax
from jax.experimental import pallas as pl
from jax.experimental.pallas import tpu as pltpu
from jax.experimental.pallas import tpu_sc as plsc

B = 16384
N_COL = 26
VOCAB = 100000
D = 64
ROWS = N_COL * D

NC = 2
NS = 16
L = 16
NW = NC * NS
D_PER_W = D // NW

OCHUNK = 2048
NCH = B // OCHUNK
UNROLL = 8
G_PER_CH = OCHUNK // (UNROLL * L)


def _make_pe(max_len, d):
    position = np.arange(max_len, dtype=np.float64)[:, None]
    div_term = np.exp(np.arange(0, d, 2, dtype=np.float64) * (-np.log(10000.0) / d))
    pe = np.zeros((max_len, d), dtype=np.float32)
    pe[:, 0::2] = np.sin(position * div_term).astype(np.float32)
    pe[:, 1::2] = np.cos(position * div_term).astype(np.float32)
    return pe


_PE26 = _make_pe(N_COL, D)

_MESH = plsc.VectorSubcoreMesh(core_axis_name="c", subcore_axis_name="s")


@functools.partial(
    pl.kernel,
    mesh=_MESH,
    out_type=jax.ShapeDtypeStruct((ROWS, B), jnp.float32),
    scratch_types=[
        pltpu.VMEM((B,), jnp.int32),
        pltpu.VMEM((VOCAB,), jnp.float32),
        pltpu.VMEM((OCHUNK,), jnp.float32),
        pltpu.VMEM((OCHUNK,), jnp.float32),
        pltpu.SemaphoreType.DMA,
        pltpu.SemaphoreType.DMA,
        pltpu.SemaphoreType.DMA,
    ],
    compiler_params=pltpu.CompilerParams(needs_layout_passes=False),
)
def _sc_gather(idx_hbm, tab_hbm, out_hbm, idx_v, vocab_v, ob0, ob1,
               sv, so0, so1):
    wid = lax.axis_index("s") * NC + lax.axis_index("c")
    ob = [ob0, ob1]
    so = [so0, so1]

    def vrow(c, rb):
        return c * D + wid * D_PER_W + rb

    pltpu.async_copy(ob0, out_hbm.at[vrow(0, 0), pl.ds(0, OCHUNK)], so0)
    pltpu.async_copy(ob1, out_hbm.at[vrow(0, 0), pl.ds(OCHUNK, OCHUNK)], so1)
    pltpu.async_copy(tab_hbm.at[vrow(0, 0)], vocab_v, sv)

    def column(c, carry):
        pltpu.sync_copy(idx_hbm.at[c], idx_v)
        cn = jnp.minimum(c + 1, N_COL - 1)

        for rb in range(D_PER_W):
            row = vrow(c, rb)
            pltpu.make_async_copy(tab_hbm.at[row], vocab_v, sv).wait()
            for j in range(NCH):
                cb = j % 2
                base = j * OCHUNK
                pltpu.make_async_copy(
                    ob[cb], out_hbm.at[row, pl.ds(base, OCHUNK)], so[cb]
                ).wait()

                def chunk_body(gg, carry2, cb=cb, base=base):
                    for u in range(UNROLL):
                        o = gg * (UNROLL * L) + u * L
                        iv = idx_v[pl.ds(base + o, L)]
                        ob[cb][pl.ds(o, L)] = plsc.load_gather(vocab_v, [iv])
                    return carry2

                lax.fori_loop(0, G_PER_CH, chunk_body, 0)
                pltpu.async_copy(
                    ob[cb], out_hbm.at[row, pl.ds(base, OCHUNK)], so[cb]
                )
            nrow = row + 1 if rb < D_PER_W - 1 else vrow(cn, 0)
            pltpu.async_copy(tab_hbm.at[nrow], vocab_v, sv)
        return carry

    lax.fori_loop(0, N_COL, column, 0)

    pltpu.make_async_copy(tab_hbm.at[0], vocab_v, sv).wait()
    pltpu.make_async_copy(ob0, out_hbm.at[0, pl.ds(0, OCHUNK)], so0).wait()
    pltpu.make_async_copy(ob1, out_hbm.at[0, pl.ds(0, OCHUNK)], so1).wait()


BC = 2048
NBC = B // BC


def _tc_body(emb_ref, w_ref, add_ref, out_ref):
    h = jnp.maximum(emb_ref[0], 0.0)
    out_ref[0] = (
        jnp.dot(w_ref[...], h, preferred_element_type=jnp.float32) + add_ref[0]
    )


def _tc_dense(emb_t, w, add3):
    return pl.pallas_call(
        _tc_body,
        grid=(N_COL, NBC),
        in_specs=[
            pl.BlockSpec((1, D, BC), lambda c, j: (c, 0, j)),
            pl.BlockSpec((D, D), lambda c, j: (0, 0)),
            pl.BlockSpec((1, D, 1), lambda c, j: (c, 0, 0)),
        ],
        out_specs=pl.BlockSpec((1, D, BC), lambda c, j: (c, 0, j)),
        out_shape=jax.ShapeDtypeStruct((N_COL, D, B), jnp.float32),
    )(emb_t, w, add3)


def kernel(x, tables, W, b):
    idx_t = x.T.astype(jnp.int32)
    tab_t = tables.transpose(0, 2, 1).reshape(ROWS, VOCAB)
    emb_t = _sc_gather(idx_t, tab_t)
    add3 = (jnp.asarray(_PE26) + b[None, :])[:, :, None]
    out_t = _tc_dense(emb_t.reshape(N_COL, D, B), W, add3)
    return out_t.transpose(2, 0, 1)


# speedup vs baseline: 2.0254x; 1.0692x over previous
import functools

import numpy as np
import jax
import jax.numpy as jnp
from jax import lax
from jax.experimental import pallas as pl
from jax.experimental.pallas import tpu as pltpu
from jax.experimental.pallas import tpu_sc as plsc

B = 16384
N_COL = 26
VOCAB = 100000
D = 64
ROWS = N_COL * D

HALF = N_COL // 2

NC = 2
NS = 16
L = 16
NW = NC * NS
D_PER_W = D // NW

OCHUNK = 2048
NCH = B // OCHUNK
UNROLL = 8
G_PER_CH = OCHUNK // (UNROLL * L)


def _make_pe(max_len, d):
    position = np.arange(max_len, dtype=np.float64)[:, None]
    div_term = np.exp(np.arange(0, d, 2, dtype=np.float64) * (-np.log(10000.0) / d))
    pe = np.zeros((max_len, d), dtype=np.float32)
    pe[:, 0::2] = np.sin(position * div_term).astype(np.float32)
    pe[:, 1::2] = np.cos(position * div_term).astype(np.float32)
    return pe


_PE26 = _make_pe(N_COL, D)

_MESH = plsc.VectorSubcoreMesh(core_axis_name="c", subcore_axis_name="s")


def _make_sc(c0, ncols):
    @functools.partial(
        pl.kernel,
        mesh=_MESH,
        out_type=jax.ShapeDtypeStruct((ncols * D, B), jnp.float32),
        scratch_types=[
            pltpu.VMEM((B,), jnp.int32),
            pltpu.VMEM((VOCAB,), jnp.float32),
            pltpu.VMEM((OCHUNK,), jnp.float32),
            pltpu.VMEM((OCHUNK,), jnp.float32),
            pltpu.SemaphoreType.DMA,
            pltpu.SemaphoreType.DMA,
            pltpu.SemaphoreType.DMA,
        ],
        compiler_params=pltpu.CompilerParams(needs_layout_passes=False),
    )
    def _sc_gather(idx_hbm, tab_hbm, out_hbm, idx_v, vocab_v, ob0, ob1,
                   sv, so0, so1):
        wid = lax.axis_index("s") * NC + lax.axis_index("c")
        ob = [ob0, ob1]
        so = [so0, so1]
        woff = wid * D_PER_W

        pltpu.async_copy(ob0, out_hbm.at[woff, pl.ds(0, OCHUNK)], so0)
        pltpu.async_copy(ob1, out_hbm.at[woff, pl.ds(OCHUNK, OCHUNK)], so1)
        pltpu.async_copy(tab_hbm.at[c0 * D + woff], vocab_v, sv)

        def column(ci, carry):
            c = c0 + ci
            pltpu.sync_copy(idx_hbm.at[c], idx_v)
            ci_n = jnp.minimum(ci + 1, ncols - 1)

            for rb in range(D_PER_W):
                grow = c * D + woff + rb
                lrow = ci * D + woff + rb
                pltpu.make_async_copy(tab_hbm.at[grow], vocab_v, sv).wait()
                for j in range(NCH):
                    cb = j % 2
                    base = j * OCHUNK
                    pltpu.make_async_copy(
                        ob[cb], out_hbm.at[lrow, pl.ds(base, OCHUNK)], so[cb]
                    ).wait()

                    def chunk_body(gg, carry2, cb=cb, base=base):
                        for u in range(UNROLL):
                            o = gg * (UNROLL * L) + u * L
                            iv = idx_v[pl.ds(base + o, L)]
                            ob[cb][pl.ds(o, L)] = plsc.load_gather(vocab_v, [iv])
                        return carry2

                    lax.fori_loop(0, G_PER_CH, chunk_body, 0)
                    pltpu.async_copy(
                        ob[cb], out_hbm.at[lrow, pl.ds(base, OCHUNK)], so[cb]
                    )
                ngrow = grow + 1 if rb < D_PER_W - 1 else (c0 + ci_n) * D + woff
                pltpu.async_copy(tab_hbm.at[ngrow], vocab_v, sv)
            return carry

        lax.fori_loop(0, ncols, column, 0)

        pltpu.make_async_copy(tab_hbm.at[0], vocab_v, sv).wait()
        pltpu.make_async_copy(ob0, out_hbm.at[0, pl.ds(0, OCHUNK)], so0).wait()
        pltpu.make_async_copy(ob1, out_hbm.at[0, pl.ds(0, OCHUNK)], so1).wait()

    return _sc_gather


_SC_A = _make_sc(0, HALF)
_SC_B = _make_sc(HALF, N_COL - HALF)


BC = 2048
NBC = B // BC


def _tc_body_first(emb_ref, w_ref, add_ref, out_ref):
    h = jnp.maximum(emb_ref[0], 0.0)
    out_ref[0] = (
        jnp.dot(w_ref[...], h, preferred_element_type=jnp.float32) + add_ref[0]
    )


def _tc_body_second(emb_ref, w_ref, add_ref, prev_ref, out_ref):
    del prev_ref
    h = jnp.maximum(emb_ref[0], 0.0)
    out_ref[0] = (
        jnp.dot(w_ref[...], h, preferred_element_type=jnp.float32) + add_ref[0]
    )


def _tc_dense_first(emb_half, w, add_half):
    return pl.pallas_call(
        _tc_body_first,
        grid=(HALF, NBC),
        in_specs=[
            pl.BlockSpec((1, D, BC), lambda c, j: (c, 0, j)),
            pl.BlockSpec((D, D), lambda c, j: (0, 0)),
            pl.BlockSpec((1, D, 1), lambda c, j: (c, 0, 0)),
        ],
        out_specs=pl.BlockSpec((1, D, BC), lambda c, j: (c, 0, j)),
        out_shape=jax.ShapeDtypeStruct((N_COL, D, B), jnp.float32),
    )(emb_half, w, add_half)


def _tc_dense_second(emb_half, w, add_half, prev):
    return pl.pallas_call(
        _tc_body_second,
        grid=(N_COL - HALF, NBC),
        in_specs=[
            pl.BlockSpec((1, D, BC), lambda c, j: (c, 0, j)),
            pl.BlockSpec((D, D), lambda c, j: (0, 0)),
            pl.BlockSpec((1, D, 1), lambda c, j: (c, 0, 0)),
            pl.BlockSpec(memory_space=pl.ANY),
        ],
        out_specs=pl.BlockSpec((1, D, BC), lambda c, j: (c + HALF, 0, j)),
        out_shape=jax.ShapeDtypeStruct((N_COL, D, B), jnp.float32),
        input_output_aliases={3: 0},
    )(emb_half, w, add_half, prev)


def kernel(x, tables, W, b):
    idx_t = x.T.astype(jnp.int32)
    tab_t = tables.transpose(0, 2, 1).reshape(ROWS, VOCAB)
    add3 = (jnp.asarray(_PE26) + b[None, :])[:, :, None]

    emb_a = _SC_A(idx_t, tab_t)
    emb_b = _SC_B(idx_t, tab_t)
    out1 = _tc_dense_first(
        emb_a.reshape(HALF, D, B), W, add3[:HALF]
    )
    out2 = _tc_dense_second(
        emb_b.reshape(N_COL - HALF, D, B), W, add3[HALF:], out1
    )
    return out2.transpose(2, 0, 1)


# speedup vs baseline: 2.5501x; 1.2591x over previous
import functools

import numpy as np
import jax
import jax.numpy as jnp
from jax import lax
from jax.experimental import pallas as pl
from jax.experimental.pallas import tpu as pltpu
from jax.experimental.pallas import tpu_sc as plsc

B = 16384
N_COL = 26
VOCAB = 100000
D = 64
ROWS = N_COL * D

HALF = N_COL // 2

NC = 2
NS = 16
L = 16
NW = NC * NS
D_PER_W = D // NW

OCHUNK = 2048
NCH = B // OCHUNK
UNROLL = 8
G_PER_CH = OCHUNK // (UNROLL * L)


def _make_pe(max_len, d):
    position = np.arange(max_len, dtype=np.float64)[:, None]
    div_term = np.exp(np.arange(0, d, 2, dtype=np.float64) * (-np.log(10000.0) / d))
    pe = np.zeros((max_len, d), dtype=np.float32)
    pe[:, 0::2] = np.sin(position * div_term).astype(np.float32)
    pe[:, 1::2] = np.cos(position * div_term).astype(np.float32)
    return pe


_PE26 = _make_pe(N_COL, D)

_MESH = plsc.VectorSubcoreMesh(core_axis_name="c", subcore_axis_name="s")


def _make_sc(c0, ncols):
    @functools.partial(
        pl.kernel,
        mesh=_MESH,
        out_type=jax.ShapeDtypeStruct((ncols * D, B), jnp.float32),
        scratch_types=[
            pltpu.VMEM((B,), jnp.int32),
            pltpu.VMEM((VOCAB,), jnp.float32),
            pltpu.VMEM((OCHUNK,), jnp.float32),
            pltpu.VMEM((OCHUNK,), jnp.float32),
            pltpu.SemaphoreType.DMA,
            pltpu.SemaphoreType.DMA,
            pltpu.SemaphoreType.DMA,
        ],
        compiler_params=pltpu.CompilerParams(needs_layout_passes=False),
    )
    def _sc_gather(idx_hbm, tab_hbm, out_hbm, idx_v, vocab_v, ob0, ob1,
                   sv, so0, so1):
        wid = lax.axis_index("s") * NC + lax.axis_index("c")
        ob = [ob0, ob1]
        so = [so0, so1]
        woff = wid * D_PER_W

        pltpu.async_copy(ob0, out_hbm.at[woff, pl.ds(0, OCHUNK)], so0)
        pltpu.async_copy(ob1, out_hbm.at[woff, pl.ds(OCHUNK, OCHUNK)], so1)
        pltpu.async_copy(tab_hbm.at[c0 * D + woff], vocab_v, sv)

        def column(ci, carry):
            c = c0 + ci
            pltpu.sync_copy(idx_hbm.at[c], idx_v)
            ci_n = jnp.minimum(ci + 1, ncols - 1)

            for rb in range(D_PER_W):
                grow = c * D + woff + rb
                lrow = ci * D + woff + rb
                pltpu.make_async_copy(tab_hbm.at[grow], vocab_v, sv).wait()
                for j in range(NCH):
                    cb = j % 2
                    base = j * OCHUNK
                    pltpu.make_async_copy(
                        ob[cb], out_hbm.at[lrow, pl.ds(base, OCHUNK)], so[cb]
                    ).wait()

                    def chunk_body(gg, carry2, cb=cb, base=base):
                        offs = [gg * (UNROLL * L) + u * L for u in range(UNROLL)]
                        ivs = [idx_v[pl.ds(base + o, L)] for o in offs]
                        vals = [plsc.load_gather(vocab_v, [iv]) for iv in ivs]
                        for o, v in zip(offs, vals):
                            ob[cb][pl.ds(o, L)] = v
                        return carry2

                    lax.fori_loop(0, G_PER_CH, chunk_body, 0)
                    pltpu.async_copy(
                        ob[cb], out_hbm.at[lrow, pl.ds(base, OCHUNK)], so[cb]
                    )
                ngrow = grow + 1 if rb < D_PER_W - 1 else (c0 + ci_n) * D + woff
                pltpu.async_copy(tab_hbm.at[ngrow], vocab_v, sv)
            return carry

        lax.fori_loop(0, ncols, column, 0)

        pltpu.make_async_copy(tab_hbm.at[0], vocab_v, sv).wait()
        pltpu.make_async_copy(ob0, out_hbm.at[0, pl.ds(0, OCHUNK)], so0).wait()
        pltpu.make_async_copy(ob1, out_hbm.at[0, pl.ds(0, OCHUNK)], so1).wait()

    return _sc_gather


_SC_A = _make_sc(0, HALF)
_SC_B = _make_sc(HALF, N_COL - HALF)


BC = 2048
NBC = B // BC


def _tc_body_first(emb_ref, w_ref, add_ref, out_ref):
    h = jnp.maximum(emb_ref[0], 0.0)
    out_ref[0] = (
        jnp.dot(w_ref[...], h, preferred_element_type=jnp.float32) + add_ref[0]
    )


def _tc_body_second(emb_ref, w_ref, add_ref, prev_ref, out_ref):
    del prev_ref
    h = jnp.maximum(emb_ref[0], 0.0)
    out_ref[0] = (
        jnp.dot(w_ref[...], h, preferred_element_type=jnp.float32) + add_ref[0]
    )


def _tc_dense_first(emb_half, w, add_half):
    return pl.pallas_call(
        _tc_body_first,
        grid=(HALF, NBC),
        in_specs=[
            pl.BlockSpec((1, D, BC), lambda c, j: (c, 0, j)),
            pl.BlockSpec((D, D), lambda c, j: (0, 0)),
            pl.BlockSpec((1, D, 1), lambda c, j: (c, 0, 0)),
        ],
        out_specs=pl.BlockSpec((1, D, BC), lambda c, j: (c, 0, j)),
        out_shape=jax.ShapeDtypeStruct((N_COL, D, B), jnp.float32),
    )(emb_half, w, add_half)


def _tc_dense_second(emb_half, w, add_half, prev):
    return pl.pallas_call(
        _tc_body_second,
        grid=(N_COL - HALF, NBC),
        in_specs=[
            pl.BlockSpec((1, D, BC), lambda c, j: (c, 0, j)),
            pl.BlockSpec((D, D), lambda c, j: (0, 0)),
            pl.BlockSpec((1, D, 1), lambda c, j: (c, 0, 0)),
            pl.BlockSpec(memory_space=pl.ANY),
        ],
        out_specs=pl.BlockSpec((1, D, BC), lambda c, j: (c + HALF, 0, j)),
        out_shape=jax.ShapeDtypeStruct((N_COL, D, B), jnp.float32),
        input_output_aliases={3: 0},
    )(emb_half, w, add_half, prev)


def kernel(x, tables, W, b):
    idx_t = x.T.astype(jnp.int32)
    tab_t = tables.transpose(0, 2, 1).reshape(ROWS, VOCAB)
    add3 = (jnp.asarray(_PE26) + b[None, :])[:, :, None]

    emb_a = _SC_A(idx_t, tab_t)
    emb_b = _SC_B(idx_t, tab_t)
    out1 = _tc_dense_first(
        emb_a.reshape(HALF, D, B), W, add3[:HALF]
    )
    out2 = _tc_dense_second(
        emb_b.reshape(N_COL - HALF, D, B), W, add3[HALF:], out1
    )
    return out2.transpose(2, 0, 1)


# speedup vs baseline: 2.6419x; 1.0360x over previous
import functools

import numpy as np
import jax
import jax.numpy as jnp
from jax import lax
from jax.experimental import pallas as pl
from jax.experimental.pallas import tpu as pltpu
from jax.experimental.pallas import tpu_sc as plsc

B = 16384
N_COL = 26
VOCAB = 100000
D = 64
ROWS = N_COL * D

HALF = N_COL // 2

NC = 2
NS = 16
L = 16
NW = NC * NS
D_PER_W = D // NW

OCHUNK = 2048
NCH = B // OCHUNK
UNROLL = 8
G_PER_CH = OCHUNK // (UNROLL * L)


def _make_pe(max_len, d):
    position = np.arange(max_len, dtype=np.float64)[:, None]
    div_term = np.exp(np.arange(0, d, 2, dtype=np.float64) * (-np.log(10000.0) / d))
    pe = np.zeros((max_len, d), dtype=np.float32)
    pe[:, 0::2] = np.sin(position * div_term).astype(np.float32)
    pe[:, 1::2] = np.cos(position * div_term).astype(np.float32)
    return pe


_PE26 = _make_pe(N_COL, D)

_MESH = plsc.VectorSubcoreMesh(core_axis_name="c", subcore_axis_name="s")


def _make_sc(c0, ncols):
    @functools.partial(
        pl.kernel,
        mesh=_MESH,
        out_type=jax.ShapeDtypeStruct((ncols * D, B), jnp.float32),
        scratch_types=[
            pltpu.VMEM((B,), jnp.int32),
            pltpu.VMEM((VOCAB,), jnp.float32),
            pltpu.VMEM((OCHUNK,), jnp.float32),
            pltpu.VMEM((OCHUNK,), jnp.float32),
            pltpu.SemaphoreType.DMA,
            pltpu.SemaphoreType.DMA,
            pltpu.SemaphoreType.DMA,
        ],
        compiler_params=pltpu.CompilerParams(needs_layout_passes=False),
    )
    def _sc_gather(idx_hbm, tab_hbm, out_hbm, idx_v, vocab_v, ob0, ob1,
                   sv, so0, so1):
        wid = lax.axis_index("s") * NC + lax.axis_index("c")
        ob = [ob0, ob1]
        so = [so0, so1]
        woff = wid * D_PER_W

        pltpu.async_copy(ob0, out_hbm.at[woff, pl.ds(0, OCHUNK)], so0)
        pltpu.async_copy(ob1, out_hbm.at[woff, pl.ds(OCHUNK, OCHUNK)], so1)
        pltpu.async_copy(tab_hbm.at[c0 * D + woff], vocab_v, sv)

        def column(ci, carry):
            c = c0 + ci
            pltpu.sync_copy(idx_hbm.at[c], idx_v)
            ci_n = jnp.minimum(ci + 1, ncols - 1)

            for rb in range(D_PER_W):
                grow = c * D + woff + rb
                lrow = ci * D + woff + rb
                pltpu.make_async_copy(tab_hbm.at[grow], vocab_v, sv).wait()
                for j in range(NCH):
                    cb = j % 2
                    base = j * OCHUNK
                    pltpu.make_async_copy(
                        ob[cb], out_hbm.at[lrow, pl.ds(base, OCHUNK)], so[cb]
                    ).wait()

                    def chunk_body(gg, carry2, cb=cb, base=base):
                        offs = [gg * (UNROLL * L) + u * L for u in range(UNROLL)]
                        ivs = [idx_v[pl.ds(base + o, L)] for o in offs]
                        vals = [plsc.load_gather(vocab_v, [iv]) for iv in ivs]
                        for o, v in zip(offs, vals):
                            ob[cb][pl.ds(o, L)] = v
                        return carry2

                    lax.fori_loop(0, G_PER_CH, chunk_body, 0)
                    pltpu.async_copy(
                        ob[cb], out_hbm.at[lrow, pl.ds(base, OCHUNK)], so[cb]
                    )
                ngrow = grow + 1 if rb < D_PER_W - 1 else (c0 + ci_n) * D + woff
                pltpu.async_copy(tab_hbm.at[ngrow], vocab_v, sv)
            return carry

        lax.fori_loop(0, ncols, column, 0)

        pltpu.make_async_copy(tab_hbm.at[0], vocab_v, sv).wait()
        pltpu.make_async_copy(ob0, out_hbm.at[0, pl.ds(0, OCHUNK)], so0).wait()
        pltpu.make_async_copy(ob1, out_hbm.at[0, pl.ds(0, OCHUNK)], so1).wait()

    return _sc_gather


_SC_A = _make_sc(0, HALF)
_SC_B = _make_sc(HALF, N_COL - HALF)


BC = 4096
NBC = B // BC


def _tc_body_first(emb_ref, w_ref, add_ref, out_ref):
    h = jnp.maximum(emb_ref[0], 0.0)
    out_ref[0] = (
        jnp.dot(w_ref[...], h, preferred_element_type=jnp.float32) + add_ref[0]
    )


def _tc_body_second(emb_ref, w_ref, add_ref, prev_ref, out_ref):
    del prev_ref
    h = jnp.maximum(emb_ref[0], 0.0)
    out_ref[0] = (
        jnp.dot(w_ref[...], h, preferred_element_type=jnp.float32) + add_ref[0]
    )


def _tc_dense_first(emb_half, w, add_half):
    return pl.pallas_call(
        _tc_body_first,
        grid=(HALF, NBC),
        in_specs=[
            pl.BlockSpec((1, D, BC), lambda c, j: (c, 0, j)),
            pl.BlockSpec((D, D), lambda c, j: (0, 0)),
            pl.BlockSpec((1, D, 1), lambda c, j: (c, 0, 0)),
        ],
        out_specs=pl.BlockSpec((1, D, BC), lambda c, j: (c, 0, j)),
        out_shape=jax.ShapeDtypeStruct((N_COL, D, B), jnp.float32),
    )(emb_half, w, add_half)


def _tc_dense_second(emb_half, w, add_half, prev):
    return pl.pallas_call(
        _tc_body_second,
        grid=(N_COL - HALF, NBC),
        in_specs=[
            pl.BlockSpec((1, D, BC), lambda c, j: (c, 0, j)),
            pl.BlockSpec((D, D), lambda c, j: (0, 0)),
            pl.BlockSpec((1, D, 1), lambda c, j: (c, 0, 0)),
            pl.BlockSpec(memory_space=pl.ANY),
        ],
        out_specs=pl.BlockSpec((1, D, BC), lambda c, j: (c + HALF, 0, j)),
        out_shape=jax.ShapeDtypeStruct((N_COL, D, B), jnp.float32),
        input_output_aliases={3: 0},
    )(emb_half, w, add_half, prev)


def kernel(x, tables, W, b):
    idx_t = x.T.astype(jnp.int32)
    tab_t = tables.transpose(0, 2, 1).reshape(ROWS, VOCAB)
    add3 = (jnp.asarray(_PE26) + b[None, :])[:, :, None]

    emb_a = _SC_A(idx_t, tab_t)
    emb_b = _SC_B(idx_t, tab_t)
    out1 = _tc_dense_first(
        emb_a.reshape(HALF, D, B), W, add3[:HALF]
    )
    out2 = _tc_dense_second(
        emb_b.reshape(N_COL - HALF, D, B), W, add3[HALF:], out1
    )
    return out2.transpose(2, 0, 1)


# speedup vs baseline: 2.6664x; 1.0093x over previous
import functools

import numpy as np
import jax
import jax.numpy as jnp
from jax import lax
from jax.experimental import pallas as pl
from jax.experimental.pallas import tpu as pltpu
from jax.experimental.pallas import tpu_sc as plsc

B = 16384
N_COL = 26
VOCAB = 100000
D = 64
ROWS = N_COL * D

HALF = N_COL // 2

NC = 2
NS = 16
L = 16
NW = NC * NS
D_PER_W = D // NW

OCHUNK = 4096
NCH = B // OCHUNK
UNROLL = 16
G_PER_CH = OCHUNK // (UNROLL * L)


def _make_pe(max_len, d):
    position = np.arange(max_len, dtype=np.float64)[:, None]
    div_term = np.exp(np.arange(0, d, 2, dtype=np.float64) * (-np.log(10000.0) / d))
    pe = np.zeros((max_len, d), dtype=np.float32)
    pe[:, 0::2] = np.sin(position * div_term).astype(np.float32)
    pe[:, 1::2] = np.cos(position * div_term).astype(np.float32)
    return pe


_PE26 = _make_pe(N_COL, D)

_MESH = plsc.VectorSubcoreMesh(core_axis_name="c", subcore_axis_name="s")


def _make_sc(c0, ncols):
    @functools.partial(
        pl.kernel,
        mesh=_MESH,
        out_type=jax.ShapeDtypeStruct((ncols * D, B), jnp.float32),
        scratch_types=[
            pltpu.VMEM((B,), jnp.int32),
            pltpu.VMEM((VOCAB,), jnp.float32),
            pltpu.VMEM((OCHUNK,), jnp.float32),
            pltpu.VMEM((OCHUNK,), jnp.float32),
            pltpu.SemaphoreType.DMA,
            pltpu.SemaphoreType.DMA,
            pltpu.SemaphoreType.DMA,
        ],
        compiler_params=pltpu.CompilerParams(needs_layout_passes=False),
    )
    def _sc_gather(idx_hbm, tab_hbm, out_hbm, idx_v, vocab_v, ob0, ob1,
                   sv, so0, so1):
        wid = lax.axis_index("s") * NC + lax.axis_index("c")
        ob = [ob0, ob1]
        so = [so0, so1]
        woff = wid * D_PER_W

        pltpu.async_copy(ob0, out_hbm.at[woff, pl.ds(0, OCHUNK)], so0)
        pltpu.async_copy(ob1, out_hbm.at[woff, pl.ds(OCHUNK, OCHUNK)], so1)
        pltpu.async_copy(tab_hbm.at[c0 * D + woff], vocab_v, sv)

        def column(ci, carry):
            c = c0 + ci
            pltpu.sync_copy(idx_hbm.at[c], idx_v)
            ci_n = jnp.minimum(ci + 1, ncols - 1)

            for rb in range(D_PER_W):
                grow = c * D + woff + rb
                lrow = ci * D + woff + rb
                pltpu.make_async_copy(tab_hbm.at[grow], vocab_v, sv).wait()
                for j in range(NCH):
                    cb = j % 2
                    base = j * OCHUNK
                    pltpu.make_async_copy(
                        ob[cb], out_hbm.at[lrow, pl.ds(base, OCHUNK)], so[cb]
                    ).wait()

                    def chunk_body(gg, carry2, cb=cb, base=base):
                        offs = [gg * (UNROLL * L) + u * L for u in range(UNROLL)]
                        ivs = [idx_v[pl.ds(base + o, L)] for o in offs]
                        vals = [plsc.load_gather(vocab_v, [iv]) for iv in ivs]
                        for o, v in zip(offs, vals):
                            ob[cb][pl.ds(o, L)] = v
                        return carry2

                    lax.fori_loop(0, G_PER_CH, chunk_body, 0)
                    pltpu.async_copy(
                        ob[cb], out_hbm.at[lrow, pl.ds(base, OCHUNK)], so[cb]
                    )
                ngrow = grow + 1 if rb < D_PER_W - 1 else (c0 + ci_n) * D + woff
                pltpu.async_copy(tab_hbm.at[ngrow], vocab_v, sv)
            return carry

        lax.fori_loop(0, ncols, column, 0)

        pltpu.make_async_copy(tab_hbm.at[0], vocab_v, sv).wait()
        pltpu.make_async_copy(ob0, out_hbm.at[0, pl.ds(0, OCHUNK)], so0).wait()
        pltpu.make_async_copy(ob1, out_hbm.at[0, pl.ds(0, OCHUNK)], so1).wait()

    return _sc_gather


_SC_A = _make_sc(0, HALF)
_SC_B = _make_sc(HALF, N_COL - HALF)


BC = 4096
NBC = B // BC


def _tc_body_first(emb_ref, w_ref, add_ref, out_ref):
    h = jnp.maximum(emb_ref[0], 0.0)
    out_ref[0] = (
        jnp.dot(w_ref[...], h, preferred_element_type=jnp.float32) + add_ref[0]
    )


def _tc_body_second(emb_ref, w_ref, add_ref, prev_ref, out_ref):
    del prev_ref
    h = jnp.maximum(emb_ref[0], 0.0)
    out_ref[0] = (
        jnp.dot(w_ref[...], h, preferred_element_type=jnp.float32) + add_ref[0]
    )


def _tc_dense_first(emb_half, w, add_half):
    return pl.pallas_call(
        _tc_body_first,
        grid=(HALF, NBC),
        in_specs=[
            pl.BlockSpec((1, D, BC), lambda c, j: (c, 0, j)),
            pl.BlockSpec((D, D), lambda c, j: (0, 0)),
            pl.BlockSpec((1, D, 1), lambda c, j: (c, 0, 0)),
        ],
        out_specs=pl.BlockSpec((1, D, BC), lambda c, j: (c, 0, j)),
        out_shape=jax.ShapeDtypeStruct((N_COL, D, B), jnp.float32),
    )(emb_half, w, add_half)


def _tc_dense_second(emb_half, w, add_half, prev):
    return pl.pallas_call(
        _tc_body_second,
        grid=(N_COL - HALF, NBC),
        in_specs=[
            pl.BlockSpec((1, D, BC), lambda c, j: (c, 0, j)),
            pl.BlockSpec((D, D), lambda c, j: (0, 0)),
            pl.BlockSpec((1, D, 1), lambda c, j: (c, 0, 0)),
            pl.BlockSpec(memory_space=pl.ANY),
        ],
        out_specs=pl.BlockSpec((1, D, BC), lambda c, j: (c + HALF, 0, j)),
        out_shape=jax.ShapeDtypeStruct((N_COL, D, B), jnp.float32),
        input_output_aliases={3: 0},
    )(emb_half, w, add_half, prev)


def kernel(x, tables, W, b):
    idx_t = x.T.astype(jnp.int32)
    tab_t = tables.transpose(0, 2, 1).reshape(ROWS, VOCAB)
    add3 = (jnp.asarray(_PE26) + b[None, :])[:, :, None]

    emb_a = _SC_A(idx_t, tab_t)
    emb_b = _SC_B(idx_t, tab_t)
    out1 = _tc_dense_first(
        emb_a.reshape(HALF, D, B), W, add3[:HALF]
    )
    out2 = _tc_dense_second(
        emb_b.reshape(N_COL - HALF, D, B), W, add3[HALF:], out1
    )
    return out2.transpose(2, 0, 1)


# speedup vs baseline: 2.7562x; 1.0337x over previous
import functools

import numpy as np
import jax
import jax.numpy as jnp
from jax import lax
from jax.experimental import pallas as pl
from jax.experimental.pallas import tpu as pltpu
from jax.experimental.pallas import tpu_sc as plsc

B = 16384
N_COL = 26
VOCAB = 100000
D = 64
ROWS = N_COL * D

HALF = N_COL // 2

NC = 2
NS = 16
L = 16
NW = NC * NS
D_PER_W = D // NW

OCHUNK = 4096
NCH = B // OCHUNK
UNROLL = 16
G_PER_CH = OCHUNK // (UNROLL * L)


def _make_pe(max_len, d):
    position = np.arange(max_len, dtype=np.float64)[:, None]
    div_term = np.exp(np.arange(0, d, 2, dtype=np.float64) * (-np.log(10000.0) / d))
    pe = np.zeros((max_len, d), dtype=np.float32)
    pe[:, 0::2] = np.sin(position * div_term).astype(np.float32)
    pe[:, 1::2] = np.cos(position * div_term).astype(np.float32)
    return pe


_PE26 = _make_pe(N_COL, D)

_MESH = plsc.VectorSubcoreMesh(core_axis_name="c", subcore_axis_name="s")


def _make_sc(c0, ncols):
    @functools.partial(
        pl.kernel,
        mesh=_MESH,
        out_type=jax.ShapeDtypeStruct((ncols * D, B), jnp.float32),
        scratch_types=[
            pltpu.VMEM((B,), jnp.int32),
            pltpu.VMEM((VOCAB,), jnp.float32),
            pltpu.VMEM((OCHUNK,), jnp.float32),
            pltpu.VMEM((OCHUNK,), jnp.float32),
            pltpu.SemaphoreType.DMA,
            pltpu.SemaphoreType.DMA,
            pltpu.SemaphoreType.DMA,
        ],
        compiler_params=pltpu.CompilerParams(needs_layout_passes=False),
    )
    def _sc_gather(idx_hbm, tab_hbm, out_hbm, idx_v, vocab_v, ob0, ob1,
                   sv, so0, so1):
        wid = lax.axis_index("s") * NC + lax.axis_index("c")
        ob = [ob0, ob1]
        so = [so0, so1]
        woff = wid * D_PER_W

        pltpu.async_copy(ob0, out_hbm.at[woff, pl.ds(0, OCHUNK)], so0)
        pltpu.async_copy(ob1, out_hbm.at[woff, pl.ds(OCHUNK, OCHUNK)], so1)
        pltpu.async_copy(tab_hbm.at[c0 * D + woff], vocab_v, sv)

        def column(ci, carry):
            c = c0 + ci
            pltpu.sync_copy(idx_hbm.at[c], idx_v)
            ci_n = jnp.minimum(ci + 1, ncols - 1)

            for rb in range(D_PER_W):
                grow = c * D + woff + rb
                lrow = ci * D + woff + rb
                pltpu.make_async_copy(tab_hbm.at[grow], vocab_v, sv).wait()
                for j in range(NCH):
                    cb = j % 2
                    base = j * OCHUNK
                    pltpu.make_async_copy(
                        ob[cb], out_hbm.at[lrow, pl.ds(base, OCHUNK)], so[cb]
                    ).wait()

                    def chunk_body(gg, carry2, cb=cb, base=base):
                        offs = [gg * (UNROLL * L) + u * L for u in range(UNROLL)]
                        ivs = [idx_v[pl.ds(base + o, L)] for o in offs]
                        vals = [plsc.load_gather(vocab_v, [iv]) for iv in ivs]
                        for o, v in zip(offs, vals):
                            ob[cb][pl.ds(o, L)] = v
                        return carry2

                    lax.fori_loop(0, G_PER_CH, chunk_body, 0)
                    pltpu.async_copy(
                        ob[cb], out_hbm.at[lrow, pl.ds(base, OCHUNK)], so[cb]
                    )
                ngrow = grow + 1 if rb < D_PER_W - 1 else (c0 + ci_n) * D + woff
                pltpu.async_copy(tab_hbm.at[ngrow], vocab_v, sv)
            return carry

        lax.fori_loop(0, ncols, column, 0)

        pltpu.make_async_copy(tab_hbm.at[0], vocab_v, sv).wait()
        pltpu.make_async_copy(ob0, out_hbm.at[0, pl.ds(0, OCHUNK)], so0).wait()
        pltpu.make_async_copy(ob1, out_hbm.at[0, pl.ds(0, OCHUNK)], so1).wait()

    return _sc_gather


_SC_A = _make_sc(0, HALF)
_SC_B = _make_sc(HALF, N_COL - HALF)


BC = 8192
NBC = B // BC


def _tc_body_first(emb_ref, w_ref, add_ref, out_ref):
    h = jnp.maximum(emb_ref[0], 0.0)
    out_ref[0] = (
        jnp.dot(w_ref[...], h, preferred_element_type=jnp.float32) + add_ref[0]
    )


def _tc_body_second(emb_ref, w_ref, add_ref, prev_ref, out_ref):
    del prev_ref
    h = jnp.maximum(emb_ref[0], 0.0)
    out_ref[0] = (
        jnp.dot(w_ref[...], h, preferred_element_type=jnp.float32) + add_ref[0]
    )


def _tc_dense_first(emb_half, w, add_half):
    return pl.pallas_call(
        _tc_body_first,
        grid=(HALF, NBC),
        in_specs=[
            pl.BlockSpec((1, D, BC), lambda c, j: (c, 0, j)),
            pl.BlockSpec((D, D), lambda c, j: (0, 0)),
            pl.BlockSpec((1, D, 1), lambda c, j: (c, 0, 0)),
        ],
        out_specs=pl.BlockSpec((1, D, BC), lambda c, j: (c, 0, j)),
        out_shape=jax.ShapeDtypeStruct((N_COL, D, B), jnp.float32),
    )(emb_half, w, add_half)


def _tc_dense_second(emb_half, w, add_half, prev):
    return pl.pallas_call(
        _tc_body_second,
        grid=(N_COL - HALF, NBC),
        in_specs=[
            pl.BlockSpec((1, D, BC), lambda c, j: (c, 0, j)),
            pl.BlockSpec((D, D), lambda c, j: (0, 0)),
            pl.BlockSpec((1, D, 1), lambda c, j: (c, 0, 0)),
            pl.BlockSpec(memory_space=pl.ANY),
        ],
        out_specs=pl.BlockSpec((1, D, BC), lambda c, j: (c + HALF, 0, j)),
        out_shape=jax.ShapeDtypeStruct((N_COL, D, B), jnp.float32),
        input_output_aliases={3: 0},
    )(emb_half, w, add_half, prev)


def kernel(x, tables, W, b):
    idx_t = x.T.astype(jnp.int32)
    tab_t = tables.transpose(0, 2, 1).reshape(ROWS, VOCAB)
    add3 = (jnp.asarray(_PE26) + b[None, :])[:, :, None]

    emb_a = _SC_A(idx_t, tab_t)
    emb_b = _SC_B(idx_t, tab_t)
    out1 = _tc_dense_first(
        emb_a.reshape(HALF, D, B), W, add3[:HALF]
    )
    out2 = _tc_dense_second(
        emb_b.reshape(N_COL - HALF, D, B), W, add3[HALF:], out1
    )
    return out2.transpose(2, 0, 1)


# speedup vs baseline: 2.7734x; 1.0062x over previous
import functools

import numpy as np
import jax
import jax.numpy as jnp
from jax import lax
from jax.experimental import pallas as pl
from jax.experimental.pallas import tpu as pltpu
from jax.experimental.pallas import tpu_sc as plsc

B = 16384
N_COL = 26
VOCAB = 100000
D = 64
ROWS = N_COL * D

HALF = N_COL // 2

NC = 2
NS = 16
L = 16
NW = NC * NS
D_PER_W = D // NW

OCHUNK = 4096
NCH = B // OCHUNK
UNROLL = 16
G_PER_CH = OCHUNK // (UNROLL * L)


def _make_pe(max_len, d):
    position = np.arange(max_len, dtype=np.float64)[:, None]
    div_term = np.exp(np.arange(0, d, 2, dtype=np.float64) * (-np.log(10000.0) / d))
    pe = np.zeros((max_len, d), dtype=np.float32)
    pe[:, 0::2] = np.sin(position * div_term).astype(np.float32)
    pe[:, 1::2] = np.cos(position * div_term).astype(np.float32)
    return pe


_PE26 = _make_pe(N_COL, D)

_MESH = plsc.VectorSubcoreMesh(core_axis_name="c", subcore_axis_name="s")


def _make_sc(c0, ncols):
    @functools.partial(
        pl.kernel,
        mesh=_MESH,
        out_type=jax.ShapeDtypeStruct((ncols * D, B), jnp.float32),
        scratch_types=[
            pltpu.VMEM((B,), jnp.int32),
            pltpu.VMEM((VOCAB,), jnp.float32),
            pltpu.VMEM((OCHUNK,), jnp.float32),
            pltpu.VMEM((OCHUNK,), jnp.float32),
            pltpu.SemaphoreType.DMA,
            pltpu.SemaphoreType.DMA,
            pltpu.SemaphoreType.DMA,
        ],
        compiler_params=pltpu.CompilerParams(needs_layout_passes=False),
    )
    def _sc_gather(idx_hbm, tab_hbm, out_hbm, idx_v, vocab_v, ob0, ob1,
                   sv, so0, so1):
        wid = lax.axis_index("s") * NC + lax.axis_index("c")
        ob = [ob0, ob1]
        so = [so0, so1]
        woff = wid * D_PER_W

        pltpu.async_copy(ob0, out_hbm.at[woff, pl.ds(0, OCHUNK)], so0)
        pltpu.async_copy(ob1, out_hbm.at[woff, pl.ds(OCHUNK, OCHUNK)], so1)
        pltpu.async_copy(tab_hbm.at[c0 * D + woff], vocab_v, sv)

        def column(ci, carry):
            c = c0 + ci
            pltpu.sync_copy(idx_hbm.at[c], idx_v)
            ci_n = jnp.minimum(ci + 1, ncols - 1)

            for rb in range(D_PER_W):
                grow = c * D + woff + rb
                lrow = ci * D + woff + rb
                pltpu.make_async_copy(tab_hbm.at[grow], vocab_v, sv).wait()
                for j in range(NCH):
                    cb = j % 2
                    base = j * OCHUNK
                    pltpu.make_async_copy(
                        ob[cb], out_hbm.at[lrow, pl.ds(base, OCHUNK)], so[cb]
                    ).wait()

                    def chunk_body(gg, carry2, cb=cb, base=base):
                        offs = [gg * (UNROLL * L) + u * L for u in range(UNROLL)]
                        ivs = [idx_v[pl.ds(base + o, L)] for o in offs]
                        vals = [plsc.load_gather(vocab_v, [iv]) for iv in ivs]
                        for o, v in zip(offs, vals):
                            ob[cb][pl.ds(o, L)] = v
                        return carry2

                    lax.fori_loop(0, G_PER_CH, chunk_body, 0)
                    pltpu.async_copy(
                        ob[cb], out_hbm.at[lrow, pl.ds(base, OCHUNK)], so[cb]
                    )
                ngrow = grow + 1 if rb < D_PER_W - 1 else (c0 + ci_n) * D + woff
                pltpu.async_copy(tab_hbm.at[ngrow], vocab_v, sv)
            return carry

        lax.fori_loop(0, ncols, column, 0)

        pltpu.make_async_copy(tab_hbm.at[0], vocab_v, sv).wait()
        pltpu.make_async_copy(ob0, out_hbm.at[0, pl.ds(0, OCHUNK)], so0).wait()
        pltpu.make_async_copy(ob1, out_hbm.at[0, pl.ds(0, OCHUNK)], so1).wait()

    return _sc_gather


_SC_A = _make_sc(0, HALF)
_SC_B = _make_sc(HALF, N_COL - HALF)


BC = 16384
NBC = B // BC


def _tc_body_first(emb_ref, w_ref, add_ref, out_ref):
    h = jnp.maximum(emb_ref[0], 0.0)
    out_ref[0] = (
        jnp.dot(w_ref[...], h, preferred_element_type=jnp.float32) + add_ref[0]
    )


def _tc_body_second(emb_ref, w_ref, add_ref, prev_ref, out_ref):
    del prev_ref
    h = jnp.maximum(emb_ref[0], 0.0)
    out_ref[0] = (
        jnp.dot(w_ref[...], h, preferred_element_type=jnp.float32) + add_ref[0]
    )


def _tc_dense_first(emb_half, w, add_half):
    return pl.pallas_call(
        _tc_body_first,
        grid=(HALF, NBC),
        in_specs=[
            pl.BlockSpec((1, D, BC), lambda c, j: (c, 0, j)),
            pl.BlockSpec((D, D), lambda c, j: (0, 0)),
            pl.BlockSpec((1, D, 1), lambda c, j: (c, 0, 0)),
        ],
        out_specs=pl.BlockSpec((1, D, BC), lambda c, j: (c, 0, j)),
        out_shape=jax.ShapeDtypeStruct((N_COL, D, B), jnp.float32),
    )(emb_half, w, add_half)


def _tc_dense_second(emb_half, w, add_half, prev):
    return pl.pallas_call(
        _tc_body_second,
        grid=(N_COL - HALF, NBC),
        in_specs=[
            pl.BlockSpec((1, D, BC), lambda c, j: (c, 0, j)),
            pl.BlockSpec((D, D), lambda c, j: (0, 0)),
            pl.BlockSpec((1, D, 1), lambda c, j: (c, 0, 0)),
            pl.BlockSpec(memory_space=pl.ANY),
        ],
        out_specs=pl.BlockSpec((1, D, BC), lambda c, j: (c + HALF, 0, j)),
        out_shape=jax.ShapeDtypeStruct((N_COL, D, B), jnp.float32),
        input_output_aliases={3: 0},
    )(emb_half, w, add_half, prev)


def kernel(x, tables, W, b):
    idx_t = x.T.astype(jnp.int32)
    tab_t = tables.transpose(0, 2, 1).reshape(ROWS, VOCAB)
    add3 = (jnp.asarray(_PE26) + b[None, :])[:, :, None]

    emb_a = _SC_A(idx_t, tab_t)
    emb_b = _SC_B(idx_t, tab_t)
    out1 = _tc_dense_first(
        emb_a.reshape(HALF, D, B), W, add3[:HALF]
    )
    out2 = _tc_dense_second(
        emb_b.reshape(N_COL - HALF, D, B), W, add3[HALF:], out1
    )
    return out2.transpose(2, 0, 1)
